# baseline (speedup 1.0000x reference)
import functools

import jax
import jax.numpy as jnp
from jax import lax
from jax.experimental import pallas as pl
from jax.experimental.pallas import tpu as pltpu
from jax.experimental.pallas import tpu_sc as plsc

_VOCAB = 1000000
_D = 64
_BATCH = 4096
_HIST = 200

_NC, _NS = 2, 16
_NW = _NC * _NS
_B = _BATCH * _HIST
_K = 128
_CH = _B // (_NW * _K)
_NBUF = 4
_NGROUPS = _CH // _NBUF

_mesh = plsc.VectorSubcoreMesh(
    core_axis_name="c", subcore_axis_name="s",
    num_cores=_NC, num_subcores=_NS)


@functools.partial(
    pl.kernel,
    out_type=jax.ShapeDtypeStruct((_NW * _CH, _K, _D), jnp.float32),
    mesh=_mesh,
    scratch_types=[
        pltpu.VMEM((_CH, _K), jnp.int32),
        pltpu.VMEM((_NBUF, _K, _D), jnp.float32),
    ] + [pltpu.SemaphoreType.DMA] * _NBUF,
    compiler_params=pltpu.CompilerParams(use_tc_tiling_on_sc=False),
)
def _emb_gather(table_hbm, idx_hbm, out_hbm, idx_v, rows_v, s0, s1, s2, s3):
    sems = (s0, s1, s2, s3)
    wid = lax.axis_index("s") * _NC + lax.axis_index("c")
    base = wid * _CH

    pltpu.sync_copy(idx_hbm.at[wid], idx_v)

    def gather_desc(j, b):
        return pltpu.make_async_copy(
            table_hbm.at[idx_v.at[j]], rows_v.at[b], sems[b])

    for b in range(_NBUF):
        gather_desc(b, b).start()

    def group(g, carry):
        for b in range(_NBUF):
            j = g * _NBUF + b
            gather_desc(j, b).wait()
            pltpu.sync_copy(rows_v.at[b], out_hbm.at[base + j])
            nj = j + _NBUF

            @pl.when(nj < _CH)
            def _():
                gather_desc(nj, b).start()
        return carry

    lax.fori_loop(0, _NGROUPS, group, 0)


def kernel(x, table):
    idx = x.astype(jnp.int32).reshape(_NW, _CH, _K)
    out = _emb_gather(table, idx)
    return out.reshape(_BATCH, _HIST, _D)
